# baseline (speedup 1.0000x reference)
import functools

import jax
import jax.numpy as jnp
from jax.experimental import pallas as pl

N = 10000
E = 320000
D = 128


def _postprocess_body(agg_r_ref, agg_d_ref, agg_rc_ref, agg_dc_ref,
                      b_real_ref, a_real_ref, b_diff_ref, a_diff_ref,
                      Wb_ref, bb_ref,
                      d1_ref, d2_ref, d3_ref, d4_ref, r_ref, h_ref):
    a_r = a_real_ref[0, 0]
    a_d = a_diff_ref[0, 0]
    bb = bb_ref[0, 0]

    def prelu(v, a):
        return jnp.where(v > 0, v, a * v)

    h_real = prelu(agg_r_ref[...] + b_real_ref[...], a_r)
    h_diff = prelu(agg_d_ref[...] + b_diff_ref[...], a_d)
    h_real_c = prelu(agg_rc_ref[...] + b_real_ref[...], a_r)
    h_diff_c = prelu(agg_dc_ref[...] + b_diff_ref[...], a_d)

    r1 = jax.nn.sigmoid(jnp.mean(h_real, axis=0, keepdims=True))
    r2 = jax.nn.sigmoid(jnp.mean(h_diff, axis=0, keepdims=True))
    r_ref[...] = r1 + r2
    h_ref[...] = h_real + h_diff

    Wb = Wb_ref[...]
    v1 = jnp.dot(Wb, r1.T, preferred_element_type=jnp.float32)
    v2 = jnp.dot(Wb, r2.T, preferred_element_type=jnp.float32)
    d1_ref[...] = jnp.dot(h_diff, v1, preferred_element_type=jnp.float32) + bb
    d2_ref[...] = jnp.dot(h_real, v2, preferred_element_type=jnp.float32) + bb
    d3_ref[...] = jnp.dot(h_diff_c, v1, preferred_element_type=jnp.float32) + bb
    d4_ref[...] = jnp.dot(h_real_c, v2, preferred_element_type=jnp.float32) + bb


@functools.partial(jax.jit, static_argnames=())
def _postprocess(agg_r, agg_d, agg_rc, agg_dc, b_real, a_real, b_diff, a_diff,
                 Wb, bb):
    out_shapes = (
        jax.ShapeDtypeStruct((N, 1), jnp.float32),
        jax.ShapeDtypeStruct((N, 1), jnp.float32),
        jax.ShapeDtypeStruct((N, 1), jnp.float32),
        jax.ShapeDtypeStruct((N, 1), jnp.float32),
        jax.ShapeDtypeStruct((1, D), jnp.float32),
        jax.ShapeDtypeStruct((N, D), jnp.float32),
    )
    return pl.pallas_call(
        _postprocess_body,
        out_shape=out_shapes,
    )(agg_r, agg_d, agg_rc, agg_dc,
      b_real.reshape(1, D), a_real.reshape(1, 1),
      b_diff.reshape(1, D), a_diff.reshape(1, 1),
      Wb, bb.reshape(1, 1))


def kernel(x, edge_index, diff_edge_index, diff_edge_weight, corrupted_idx,
           W_real, b_real, a_real, W_diff, b_diff, a_diff, Wb, bb):
    row_r, col_r = edge_index[0], edge_index[1]
    row_d, col_d = diff_edge_index[0], diff_edge_index[1]

    deg_r = jax.ops.segment_sum(jnp.ones((E,), jnp.float32), col_r,
                                num_segments=N) + 1.0
    deg_d = jax.ops.segment_sum(diff_edge_weight, col_d,
                                num_segments=N) + 1.0
    dinv_r = jax.lax.rsqrt(deg_r)
    dinv_d = jax.lax.rsqrt(deg_d)

    norm_r = dinv_r[row_r] * dinv_r[col_r]
    norm_d = dinv_d[row_d] * diff_edge_weight * dinv_d[col_d]

    xw_r = x @ W_real
    xw_d = x @ W_diff

    p = corrupted_idx
    row_r_c = p[row_r]
    row_d_c = p[row_d]

    def agg(norm, src_rows, col, xw, dinv, self_rows):
        out = jax.ops.segment_sum(norm[:, None] * jnp.take(xw, src_rows, axis=0),
                                  col, num_segments=N)
        return out + (dinv * dinv)[:, None] * jnp.take(xw, self_rows, axis=0)

    iota = jnp.arange(N, dtype=jnp.int32)
    agg_r = agg(norm_r, row_r, col_r, xw_r, dinv_r, iota)
    agg_d = agg(norm_d, row_d, col_d, xw_d, dinv_d, iota)
    agg_rc = agg(norm_r, row_r_c, col_r, xw_r, dinv_r, p)
    agg_dc = agg(norm_d, row_d_c, col_d, xw_d, dinv_d, p)

    d1, d2, d3, d4, r_sum, h_sum = _postprocess(
        agg_r, agg_d, agg_rc, agg_dc, b_real, a_real, b_diff, a_diff, Wb, bb)

    disc = jnp.concatenate([d1[:, 0], d2[:, 0], d3[:, 0], d4[:, 0]])
    return (disc, r_sum[0], h_sum)


# speedup vs baseline: 12.2921x; 12.2921x over previous
import functools

import jax
import jax.numpy as jnp
from jax import lax
from jax.experimental import pallas as pl
from jax.experimental.pallas import tpu as pltpu
from jax.experimental.pallas import tpu_sc as plsc

N = 10000
E = 320000
D = 128
NP = 10240
EPT = E // 16
PAD_E = 96
EPT_P = EPT + PAD_E
NCH = EPT_P // 128

_SC_PARAMS = pltpu.CompilerParams(
    needs_layout_passes=False, use_tc_tiling_on_sc=False)
_mesh = plsc.VectorSubcoreMesh(core_axis_name="c", subcore_axis_name="s")

_ZERO16 = functools.partial(jnp.zeros, (16,), jnp.float32)


@functools.partial(
    pl.kernel, mesh=_mesh,
    out_type=[
        jax.ShapeDtypeStruct((2, NP), jnp.float32),
        jax.ShapeDtypeStruct((NP, D), jnp.float32),
    ],
    scratch_types=[
        pltpu.VMEM((EPT,), jnp.int32),
        pltpu.VMEM((EPT,), jnp.float32),
        pltpu.VMEM((NP,), jnp.float32),
        pltpu.VMEM((16, 640), jnp.float32),
        pltpu.VMEM((640,), jnp.float32),
        pltpu.VMEM((4, 80), jnp.int32),
        pltpu.VMEM((80, D), jnp.float32),
        pltpu.VMEM_SHARED((16, NP), jnp.float32),
        pltpu.SemaphoreType.DMA,
    ],
    compiler_params=_SC_PARAMS,
)
def _sc_deg_xc(col_r, col_d, w_d, p_pad, x, deg_out, xc_out,
               col_v, w_v, deg_p, rbuf, dchunk, pidx, xbuf, sh_deg, sem):
    c = lax.axis_index("c")
    s = lax.axis_index("s")
    wid = c * 16 + s

    def zero_deg(i, _):
        deg_p[pl.ds(i * 16, 16)] = _ZERO16()
        return 0
    lax.fori_loop(0, NP // 16, zero_deg, 0)

    @pl.when(c == 0)
    def _():
        pltpu.sync_copy(col_r.at[pl.ds(s * EPT, EPT)], col_v)
        ones = jnp.full((16,), 1.0, jnp.float32)
        def body(i, _):
            col16 = col_v[pl.ds(i * 16, 16)]
            plsc.addupdate_scatter(deg_p, [col16], ones)
            return 0
        lax.fori_loop(0, EPT // 16, body, 0)

    @pl.when(c == 1)
    def _():
        pltpu.sync_copy(col_d.at[pl.ds(s * EPT, EPT)], col_v)
        pltpu.sync_copy(w_d.at[pl.ds(s * EPT, EPT)], w_v)
        def body(i, _):
            col16 = col_v[pl.ds(i * 16, 16)]
            w16 = w_v[pl.ds(i * 16, 16)]
            plsc.addupdate_scatter(deg_p, [col16], w16)
            return 0
        lax.fori_loop(0, EPT // 16, body, 0)

    pltpu.sync_copy(deg_p, sh_deg.at[s])
    plsc.subcore_barrier()
    pltpu.sync_copy(sh_deg.at[:, pl.ds(s * 640, 640)], rbuf)

    def red(j, _):
        sl = pl.ds(j * 16, 16)
        acc16 = rbuf[0, sl]
        for i in range(1, 16):
            acc16 = acc16 + rbuf[i, sl]
        dchunk[sl] = acc16
        return 0
    lax.fori_loop(0, 40, red, 0)
    pltpu.sync_copy(dchunk, deg_out.at[c].at[pl.ds(s * 640, 640)])

    for j in range(4):
        base = wid * 320 + j * 80
        pltpu.sync_copy(p_pad.at[pl.ds(base, 80)], pidx.at[j])
        pltpu.async_copy(x.at[pidx.at[j]], xbuf, sem).wait()
        pltpu.sync_copy(xbuf, xc_out.at[pl.ds(base, 80)])


def _tables_body(x_ref, xc_ref, wr_ref, wd_ref, degr_ref, degd_ref,
                 t1_ref, t2_ref, t3_ref, t4_ref, dvr_ref, dvd_ref):
    dvr = lax.rsqrt(degr_ref[...] + 1.0)
    dvd = lax.rsqrt(degd_ref[...] + 1.0)
    dvr_ref[...] = dvr
    dvd_ref[...] = dvd
    x = x_ref[...]
    xc = xc_ref[...]
    wr = wr_ref[...]
    wd = wd_ref[...]
    t1_ref[...] = jnp.dot(x, wr, preferred_element_type=jnp.float32) * dvr
    t2_ref[...] = jnp.dot(xc, wr, preferred_element_type=jnp.float32) * dvr
    t3_ref[...] = jnp.dot(x, wd, preferred_element_type=jnp.float32) * dvd
    t4_ref[...] = jnp.dot(xc, wd, preferred_element_type=jnp.float32) * dvd


def _tc_tables(x_pad, xc, W_real, W_diff, deg_r, deg_d):
    B = 1024
    nblk = NP // B
    row_spec = pl.BlockSpec((B, D), lambda i: (i, 0))
    col1_spec = pl.BlockSpec((B, 1), lambda i: (i, 0))
    w_spec = pl.BlockSpec((D, D), lambda i: (0, 0))
    return pl.pallas_call(
        _tables_body,
        grid=(nblk,),
        in_specs=[row_spec, row_spec, w_spec, w_spec, col1_spec, col1_spec],
        out_specs=[row_spec, row_spec, row_spec, row_spec, col1_spec, col1_spec],
        out_shape=[
            jax.ShapeDtypeStruct((NP, D), jnp.float32),
            jax.ShapeDtypeStruct((NP, D), jnp.float32),
            jax.ShapeDtypeStruct((NP, D), jnp.float32),
            jax.ShapeDtypeStruct((NP, D), jnp.float32),
            jax.ShapeDtypeStruct((NP, 1), jnp.float32),
            jax.ShapeDtypeStruct((NP, 1), jnp.float32),
        ],
    )(x_pad, xc, W_real, W_diff, deg_r, deg_d)


@functools.partial(
    pl.kernel, mesh=_mesh,
    out_type=[
        jax.ShapeDtypeStruct((NP, D), jnp.float32),
        jax.ShapeDtypeStruct((NP, D), jnp.float32),
        jax.ShapeDtypeStruct((NP, D), jnp.float32),
        jax.ShapeDtypeStruct((NP, D), jnp.float32),
    ],
    scratch_types=[
        pltpu.VMEM((1, 128), jnp.int32),
        pltpu.VMEM((1, 128), jnp.int32),
        pltpu.VMEM((128,), jnp.float32),
        pltpu.VMEM((128, D), jnp.float32),
        pltpu.VMEM((128, D), jnp.float32),
        pltpu.VMEM_SHARED((NP, D), jnp.float32),
        pltpu.SemaphoreType.DMA,
    ],
    compiler_params=_SC_PARAMS,
)
def _sc_agg(t1, t2, t3, t4, rowr, colr, rowd, cold, wpad,
            agg_r, agg_rc, agg_d, agg_dc,
            idx_row, idx_col, wbuf, gbuf, zbuf, acc, sem):
    c = lax.axis_index("c")
    s = lax.axis_index("s")

    def zrow(i, _):
        for j in range(8):
            zbuf[i, pl.ds(j * 16, 16)] = _ZERO16()
        return 0
    lax.fori_loop(0, 128, zrow, 0)

    def one_round(table, rows, cols, weights, out):
        for k in range(5):
            pltpu.sync_copy(zbuf, acc.at[pl.ds((s * 5 + k) * 128, 128)])
        plsc.subcore_barrier()

        def chunk(g, _):
            sl = pl.ds(g * 128, 128)
            pltpu.sync_copy(rows.at[s].at[sl], idx_row.at[0])
            pltpu.sync_copy(cols.at[s].at[sl], idx_col.at[0])
            pltpu.async_copy(table.at[idx_row.at[0]], gbuf, sem).wait()
            if weights is not None:
                pltpu.sync_copy(weights.at[s].at[sl], wbuf)
                def scale(i, _):
                    wsp = plsc.load_gather(
                        wbuf, [jnp.full((16,), 1, jnp.int32) * i])
                    for j in range(8):
                        jsl = pl.ds(j * 16, 16)
                        gbuf[i, jsl] = gbuf[i, jsl] * wsp
                    return 0
                lax.fori_loop(0, 128, scale, 0)
            pltpu.sync_copy(gbuf, acc.at[idx_col.at[0]], add=True)
            return 0
        lax.fori_loop(0, NCH, chunk, 0)
        plsc.subcore_barrier()
        for k in range(5):
            off = (s * 5 + k) * 128
            pltpu.sync_copy(acc.at[pl.ds(off, 128)], out.at[pl.ds(off, 128)])
        plsc.subcore_barrier()

    @pl.when(c == 0)
    def _():
        one_round(t1, rowr, colr, None, agg_r)
        one_round(t3, rowd, cold, wpad, agg_d)

    @pl.when(c == 1)
    def _():
        one_round(t2, rowr, colr, None, agg_rc)
        one_round(t4, rowd, cold, wpad, agg_dc)


_PB = 2000


def _post_a_body(aggr_ref, aggd_ref, aggrc_ref, aggdc_ref,
                 t1_ref, t2_ref, t3_ref, t4_ref, dvr_ref, dvd_ref,
                 br_ref, ar_ref, bd_ref, ad_ref,
                 hr_ref, hd_ref, hrc_ref, hdc_ref, hsum_ref, sr_ref, sd_ref):
    i = pl.program_id(0)
    a_r = ar_ref[0, 0]
    a_d = ad_ref[0, 0]
    dvr = dvr_ref[...]
    dvd = dvd_ref[...]

    def prelu(v, a):
        return jnp.where(v > 0, v, a * v)

    h_real = prelu(dvr * (aggr_ref[...] + t1_ref[...]) + br_ref[...], a_r)
    h_diff = prelu(dvd * (aggd_ref[...] + t3_ref[...]) + bd_ref[...], a_d)
    h_real_c = prelu(dvr * (aggrc_ref[...] + t2_ref[...]) + br_ref[...], a_r)
    h_diff_c = prelu(dvd * (aggdc_ref[...] + t4_ref[...]) + bd_ref[...], a_d)

    hr_ref[...] = h_real
    hd_ref[...] = h_diff
    hrc_ref[...] = h_real_c
    hdc_ref[...] = h_diff_c
    hsum_ref[...] = h_real + h_diff

    @pl.when(i == 0)
    def _():
        sr_ref[...] = jnp.zeros((1, D), jnp.float32)
        sd_ref[...] = jnp.zeros((1, D), jnp.float32)
    sr_ref[...] += jnp.sum(h_real, axis=0, keepdims=True)
    sd_ref[...] += jnp.sum(h_diff, axis=0, keepdims=True)


def _post_b_body(sr_ref, sd_ref, Wb_ref, r_ref, v1_ref, v2_ref):
    r1 = jax.nn.sigmoid(sr_ref[...] / N)
    r2 = jax.nn.sigmoid(sd_ref[...] / N)
    r_ref[...] = r1 + r2
    Wb = Wb_ref[...]
    v1_ref[...] = jnp.dot(Wb, r1.T, preferred_element_type=jnp.float32)
    v2_ref[...] = jnp.dot(Wb, r2.T, preferred_element_type=jnp.float32)


def _post_c_body(hr_ref, hd_ref, hrc_ref, hdc_ref, v1_ref, v2_ref, bb_ref,
                 d1_ref, d2_ref, d3_ref, d4_ref):
    bb = bb_ref[0, 0]
    v1 = v1_ref[...]
    v2 = v2_ref[...]
    d1_ref[...] = jnp.dot(hd_ref[...], v1, preferred_element_type=jnp.float32) + bb
    d2_ref[...] = jnp.dot(hr_ref[...], v2, preferred_element_type=jnp.float32) + bb
    d3_ref[...] = jnp.dot(hdc_ref[...], v1, preferred_element_type=jnp.float32) + bb
    d4_ref[...] = jnp.dot(hrc_ref[...], v2, preferred_element_type=jnp.float32) + bb


def _tc_post(agg_r, agg_d, agg_rc, agg_dc, t1, t2, t3, t4, dvr, dvd,
             b_real, a_real, b_diff, a_diff, Wb, bb):
    nblk = N // _PB
    row_spec = pl.BlockSpec((_PB, D), lambda i: (i, 0))
    col1_spec = pl.BlockSpec((_PB, 1), lambda i: (i, 0))
    full_spec = pl.BlockSpec((1, D), lambda i: (0, 0))
    one_spec = pl.BlockSpec((1, 1), lambda i: (0, 0))
    hr, hd, hrc, hdc, h_sum, sr, sd = pl.pallas_call(
        _post_a_body,
        grid=(nblk,),
        in_specs=[row_spec] * 8 + [col1_spec] * 2
        + [full_spec, one_spec, full_spec, one_spec],
        out_specs=[row_spec] * 5 + [full_spec, full_spec],
        out_shape=[jax.ShapeDtypeStruct((N, D), jnp.float32)] * 5
        + [jax.ShapeDtypeStruct((1, D), jnp.float32)] * 2,
    )(agg_r, agg_d, agg_rc, agg_dc, t1, t2, t3, t4, dvr, dvd,
      b_real.reshape(1, D), a_real.reshape(1, 1),
      b_diff.reshape(1, D), a_diff.reshape(1, 1))

    r_sum, v1, v2 = pl.pallas_call(
        _post_b_body,
        out_shape=[
            jax.ShapeDtypeStruct((1, D), jnp.float32),
            jax.ShapeDtypeStruct((D, 1), jnp.float32),
            jax.ShapeDtypeStruct((D, 1), jnp.float32),
        ],
    )(sr, sd, Wb)

    dcol_spec = pl.BlockSpec((_PB, 1), lambda i: (i, 0))
    vfull_spec = pl.BlockSpec((D, 1), lambda i: (0, 0))
    d1, d2, d3, d4 = pl.pallas_call(
        _post_c_body,
        grid=(nblk,),
        in_specs=[row_spec] * 4 + [vfull_spec, vfull_spec, one_spec],
        out_specs=[dcol_spec] * 4,
        out_shape=[jax.ShapeDtypeStruct((N, 1), jnp.float32)] * 4,
    )(hr, hd, hrc, hdc, v1, v2, bb.reshape(1, 1))
    return d1, d2, d3, d4, r_sum, h_sum


def kernel(x, edge_index, diff_edge_index, diff_edge_weight, corrupted_idx,
           W_real, b_real, a_real, W_diff, b_diff, a_diff, Wb, bb):
    row_r, col_r = edge_index[0], edge_index[1]
    row_d, col_d = diff_edge_index[0], diff_edge_index[1]

    def pad_idx(a):
        return jnp.pad(a.reshape(16, EPT), ((0, 0), (0, PAD_E)),
                       constant_values=N)

    rowr_p = pad_idx(row_r)
    colr_p = pad_idx(col_r)
    rowd_p = pad_idx(row_d)
    cold_p = pad_idx(col_d)
    w_p = jnp.pad(diff_edge_weight.reshape(16, EPT), ((0, 0), (0, PAD_E)))

    p_pad = jnp.pad(corrupted_idx, (0, NP - N))
    x_pad = jnp.pad(x, ((0, NP - N), (0, 0)))

    deg, xc = _sc_deg_xc(col_r, col_d, diff_edge_weight, p_pad, x)
    t1, t2, t3, t4, dvr, dvd = _tc_tables(
        x_pad, xc, W_real, W_diff,
        deg[0].reshape(NP, 1), deg[1].reshape(NP, 1))
    agg_r, agg_rc, agg_d, agg_dc = _sc_agg(
        t1, t2, t3, t4, rowr_p, colr_p, rowd_p, cold_p, w_p)

    d1, d2, d3, d4, r_sum, h_sum = _tc_post(
        agg_r[:N], agg_d[:N], agg_rc[:N], agg_dc[:N],
        t1[:N], t2[:N], t3[:N], t4[:N], dvr[:N], dvd[:N],
        b_real, a_real, b_diff, a_diff, Wb, bb)

    disc = jnp.concatenate([d1[:, 0], d2[:, 0], d3[:, 0], d4[:, 0]])
    return (disc, r_sum[0], h_sum)


# speedup vs baseline: 14.1337x; 1.1498x over previous
import functools

import jax
import jax.numpy as jnp
from jax import lax
from jax.experimental import pallas as pl
from jax.experimental.pallas import tpu as pltpu
from jax.experimental.pallas import tpu_sc as plsc

N = 10000
E = 320000
D = 128
NP = 10240
EPT = E // 16
PAD_E = 480
EPT_P = EPT + PAD_E
NCH = EPT_P // 128
BLK = 8
NBLK = NCH // BLK

_SC_PARAMS = pltpu.CompilerParams(
    needs_layout_passes=False, use_tc_tiling_on_sc=False)
_mesh = plsc.VectorSubcoreMesh(core_axis_name="c", subcore_axis_name="s")

_ZERO16 = functools.partial(jnp.zeros, (16,), jnp.float32)


@functools.partial(
    pl.kernel, mesh=_mesh,
    out_type=[
        jax.ShapeDtypeStruct((2, NP), jnp.float32),
        jax.ShapeDtypeStruct((NP, D), jnp.float32),
    ],
    scratch_types=[
        pltpu.VMEM((EPT,), jnp.int32),
        pltpu.VMEM((EPT,), jnp.float32),
        pltpu.VMEM((NP,), jnp.float32),
        pltpu.VMEM((16, 640), jnp.float32),
        pltpu.VMEM((640,), jnp.float32),
        pltpu.VMEM((4, 80), jnp.int32),
        pltpu.VMEM((80, D), jnp.float32),
        pltpu.VMEM_SHARED((16, NP), jnp.float32),
        pltpu.SemaphoreType.DMA,
    ],
    compiler_params=_SC_PARAMS,
)
def _sc_deg_xc(col_r, col_d, w_d, p_pad, x, deg_out, xc_out,
               col_v, w_v, deg_p, rbuf, dchunk, pidx, xbuf, sh_deg, sem):
    c = lax.axis_index("c")
    s = lax.axis_index("s")
    wid = c * 16 + s

    def zero_deg(i, _):
        deg_p[pl.ds(i * 16, 16)] = _ZERO16()
        return 0
    lax.fori_loop(0, NP // 16, zero_deg, 0)

    @pl.when(c == 0)
    def _():
        pltpu.sync_copy(col_r.at[pl.ds(s * EPT, EPT)], col_v)
        ones = jnp.full((16,), 1.0, jnp.float32)
        def body(i, _):
            col16 = col_v[pl.ds(i * 16, 16)]
            plsc.addupdate_scatter(deg_p, [col16], ones)
            return 0
        lax.fori_loop(0, EPT // 16, body, 0)

    @pl.when(c == 1)
    def _():
        pltpu.sync_copy(col_d.at[pl.ds(s * EPT, EPT)], col_v)
        pltpu.sync_copy(w_d.at[pl.ds(s * EPT, EPT)], w_v)
        def body(i, _):
            col16 = col_v[pl.ds(i * 16, 16)]
            w16 = w_v[pl.ds(i * 16, 16)]
            plsc.addupdate_scatter(deg_p, [col16], w16)
            return 0
        lax.fori_loop(0, EPT // 16, body, 0)

    pltpu.sync_copy(deg_p, sh_deg.at[s])
    plsc.subcore_barrier()
    pltpu.sync_copy(sh_deg.at[:, pl.ds(s * 640, 640)], rbuf)

    def red(j, _):
        sl = pl.ds(j * 16, 16)
        acc16 = rbuf[0, sl]
        for i in range(1, 16):
            acc16 = acc16 + rbuf[i, sl]
        dchunk[sl] = acc16
        return 0
    lax.fori_loop(0, 40, red, 0)
    pltpu.sync_copy(dchunk, deg_out.at[c].at[pl.ds(s * 640, 640)])

    for j in range(4):
        base = wid * 320 + j * 80
        pltpu.sync_copy(p_pad.at[pl.ds(base, 80)], pidx.at[j])
        pltpu.async_copy(x.at[pidx.at[j]], xbuf, sem).wait()
        pltpu.sync_copy(xbuf, xc_out.at[pl.ds(base, 80)])


def _tables_body(x_ref, xc_ref, wr_ref, wd_ref, degr_ref, degd_ref,
                 t1_ref, t2_ref, t3_ref, t4_ref, dvr_ref, dvd_ref):
    dvr = lax.rsqrt(degr_ref[...] + 1.0)
    dvd = lax.rsqrt(degd_ref[...] + 1.0)
    dvr_ref[...] = dvr
    dvd_ref[...] = dvd
    x = x_ref[...]
    xc = xc_ref[...]
    wr = wr_ref[...]
    wd = wd_ref[...]
    t1_ref[...] = jnp.dot(x, wr, preferred_element_type=jnp.float32) * dvr
    t2_ref[...] = jnp.dot(xc, wr, preferred_element_type=jnp.float32) * dvr
    t3_ref[...] = jnp.dot(x, wd, preferred_element_type=jnp.float32) * dvd
    t4_ref[...] = jnp.dot(xc, wd, preferred_element_type=jnp.float32) * dvd


def _tc_tables(x_pad, xc, W_real, W_diff, deg_r, deg_d):
    B = 1024
    nblk = NP // B
    row_spec = pl.BlockSpec((B, D), lambda i: (i, 0))
    col1_spec = pl.BlockSpec((B, 1), lambda i: (i, 0))
    w_spec = pl.BlockSpec((D, D), lambda i: (0, 0))
    return pl.pallas_call(
        _tables_body,
        grid=(nblk,),
        in_specs=[row_spec, row_spec, w_spec, w_spec, col1_spec, col1_spec],
        out_specs=[row_spec, row_spec, row_spec, row_spec, col1_spec, col1_spec],
        out_shape=[
            jax.ShapeDtypeStruct((NP, D), jnp.float32),
            jax.ShapeDtypeStruct((NP, D), jnp.float32),
            jax.ShapeDtypeStruct((NP, D), jnp.float32),
            jax.ShapeDtypeStruct((NP, D), jnp.float32),
            jax.ShapeDtypeStruct((NP, 1), jnp.float32),
            jax.ShapeDtypeStruct((NP, 1), jnp.float32),
        ],
    )(x_pad, xc, W_real, W_diff, deg_r, deg_d)


@functools.partial(
    pl.kernel, mesh=_mesh,
    out_type=[
        jax.ShapeDtypeStruct((NP, D), jnp.float32),
        jax.ShapeDtypeStruct((NP, D), jnp.float32),
        jax.ShapeDtypeStruct((NP, D), jnp.float32),
        jax.ShapeDtypeStruct((NP, D), jnp.float32),
    ],
    scratch_types=[
        pltpu.VMEM((2, BLK, 128), jnp.int32),
        pltpu.VMEM((2, BLK, 128), jnp.int32),
        pltpu.VMEM((2, BLK, 128), jnp.float32),
        pltpu.VMEM((128, D), jnp.float32),
        pltpu.VMEM((128, D), jnp.float32),
        pltpu.VMEM((32, D), jnp.float32),
        pltpu.VMEM_SHARED((NP, D), jnp.float32),
        pltpu.SemaphoreType.DMA((2,)),
        pltpu.SemaphoreType.DMA((2,)),
    ],
    compiler_params=_SC_PARAMS,
)
def _sc_agg(t1, t2, t3, t4, rowr, colr, rowd, cold, wpad,
            agg_r, agg_rc, agg_d, agg_dc,
            rows2, cols2, wv2, gb0, gb1, zbuf, acc, gsem, isem):
    c = lax.axis_index("c")
    s = lax.axis_index("s")
    gbufs = (gb0, gb1)

    def zrow(i, _):
        for j in range(8):
            zbuf[i, pl.ds(j * 16, 16)] = _ZERO16()
        return 0
    lax.fori_loop(0, 32, zrow, 0)

    def one_round(table, rows, cols, weights, out):
        for k in range(20):
            pltpu.sync_copy(zbuf, acc.at[pl.ds((s * 20 + k) * 32, 32)])

        def idx_descs(bb, par):
            sl = pl.ds(bb * BLK, BLK)
            ds_ = [
                pltpu.make_async_copy(rows.at[s].at[sl], rows2.at[par],
                                      isem.at[par]),
                pltpu.make_async_copy(cols.at[s].at[sl], cols2.at[par],
                                      isem.at[par]),
            ]
            if weights is not None:
                ds_.append(pltpu.make_async_copy(
                    weights.at[s].at[sl], wv2.at[par], isem.at[par]))
            return ds_

        def start_block(bb, par):
            for d_ in idx_descs(bb, par):
                d_.start()

        def wait_block(bb, par):
            for d_ in idx_descs(bb, par):
                d_.wait()

        def gather_desc(b, par, pos):
            return pltpu.make_async_copy(
                table.at[rows2.at[par, pos]], gbufs[b], gsem.at[b])

        start_block(0, 0)
        wait_block(0, 0)
        plsc.subcore_barrier()
        gather_desc(0, 0, 0).start()
        gather_desc(1, 0, 1).start()

        ones16 = jnp.full((16,), 1, jnp.int32)

        def chunk_body(bb, par, pos, b):
            g = bb * BLK + pos
            gather_desc(b, par, pos).wait()
            gbuf = gbufs[b]
            if weights is not None:
                def scale(i, _):
                    wsp = plsc.load_gather(
                        wv2, [ones16 * par, ones16 * pos, ones16 * i])
                    for j in range(8):
                        jsl = pl.ds(j * 16, 16)
                        gbuf[i, jsl] = gbuf[i, jsl] * wsp
                    return 0
                lax.fori_loop(0, 128, scale, 0)
            pltpu.sync_copy(gbuf, acc.at[cols2.at[par, pos]], add=True)
            if pos == 0:
                @pl.when(bb + 1 < NBLK)
                def _():
                    start_block(bb + 1, 1 - par)
            if pos == 6:
                @pl.when(bb + 1 < NBLK)
                def _():
                    wait_block(bb + 1, 1 - par)
            if pos < 6:
                gather_desc(b, par, pos + 2).start()
            else:
                @pl.when(bb + 1 < NBLK)
                def _():
                    gather_desc(b, 1 - par, pos - 6).start()

        def sbody(bb2, _):
            for half in range(2):
                bb = bb2 * 2 + half
                for pos in range(BLK):
                    chunk_body(bb, half, pos, pos % 2)
            return 0
        lax.fori_loop(0, NBLK // 2, sbody, 0)
        plsc.subcore_barrier()
        for k in range(5):
            off = (s * 5 + k) * 128
            pltpu.sync_copy(acc.at[pl.ds(off, 128)], out.at[pl.ds(off, 128)])
        plsc.subcore_barrier()

    @pl.when(c == 0)
    def _():
        one_round(t1, rowr, colr, None, agg_r)
        one_round(t3, rowd, cold, wpad, agg_d)

    @pl.when(c == 1)
    def _():
        one_round(t2, rowr, colr, None, agg_rc)
        one_round(t4, rowd, cold, wpad, agg_dc)


_PB = 2000


def _post_a_body(aggr_ref, aggd_ref, aggrc_ref, aggdc_ref,
                 t1_ref, t2_ref, t3_ref, t4_ref, dvr_ref, dvd_ref,
                 br_ref, ar_ref, bd_ref, ad_ref,
                 hr_ref, hd_ref, hrc_ref, hdc_ref, hsum_ref, sr_ref, sd_ref):
    i = pl.program_id(0)
    a_r = ar_ref[0, 0]
    a_d = ad_ref[0, 0]
    dvr = dvr_ref[...]
    dvd = dvd_ref[...]

    def prelu(v, a):
        return jnp.where(v > 0, v, a * v)

    h_real = prelu(dvr * (aggr_ref[...] + t1_ref[...]) + br_ref[...], a_r)
    h_diff = prelu(dvd * (aggd_ref[...] + t3_ref[...]) + bd_ref[...], a_d)
    h_real_c = prelu(dvr * (aggrc_ref[...] + t2_ref[...]) + br_ref[...], a_r)
    h_diff_c = prelu(dvd * (aggdc_ref[...] + t4_ref[...]) + bd_ref[...], a_d)

    hr_ref[...] = h_real
    hd_ref[...] = h_diff
    hrc_ref[...] = h_real_c
    hdc_ref[...] = h_diff_c
    hsum_ref[...] = h_real + h_diff

    @pl.when(i == 0)
    def _():
        sr_ref[...] = jnp.zeros((1, D), jnp.float32)
        sd_ref[...] = jnp.zeros((1, D), jnp.float32)
    sr_ref[...] += jnp.sum(h_real, axis=0, keepdims=True)
    sd_ref[...] += jnp.sum(h_diff, axis=0, keepdims=True)


def _post_b_body(sr_ref, sd_ref, Wb_ref, r_ref, v1_ref, v2_ref):
    r1 = jax.nn.sigmoid(sr_ref[...] / N)
    r2 = jax.nn.sigmoid(sd_ref[...] / N)
    r_ref[...] = r1 + r2
    Wb = Wb_ref[...]
    v1_ref[...] = jnp.dot(Wb, r1.T, preferred_element_type=jnp.float32)
    v2_ref[...] = jnp.dot(Wb, r2.T, preferred_element_type=jnp.float32)


def _post_c_body(hr_ref, hd_ref, hrc_ref, hdc_ref, v1_ref, v2_ref, bb_ref,
                 d1_ref, d2_ref, d3_ref, d4_ref):
    bb = bb_ref[0, 0]
    v1 = v1_ref[...]
    v2 = v2_ref[...]
    d1_ref[...] = jnp.dot(hd_ref[...], v1, preferred_element_type=jnp.float32) + bb
    d2_ref[...] = jnp.dot(hr_ref[...], v2, preferred_element_type=jnp.float32) + bb
    d3_ref[...] = jnp.dot(hdc_ref[...], v1, preferred_element_type=jnp.float32) + bb
    d4_ref[...] = jnp.dot(hrc_ref[...], v2, preferred_element_type=jnp.float32) + bb


def _tc_post(agg_r, agg_d, agg_rc, agg_dc, t1, t2, t3, t4, dvr, dvd,
             b_real, a_real, b_diff, a_diff, Wb, bb):
    nblk = N // _PB
    row_spec = pl.BlockSpec((_PB, D), lambda i: (i, 0))
    col1_spec = pl.BlockSpec((_PB, 1), lambda i: (i, 0))
    full_spec = pl.BlockSpec((1, D), lambda i: (0, 0))
    one_spec = pl.BlockSpec((1, 1), lambda i: (0, 0))
    hr, hd, hrc, hdc, h_sum, sr, sd = pl.pallas_call(
        _post_a_body,
        grid=(nblk,),
        in_specs=[row_spec] * 8 + [col1_spec] * 2
        + [full_spec, one_spec, full_spec, one_spec],
        out_specs=[row_spec] * 5 + [full_spec, full_spec],
        out_shape=[jax.ShapeDtypeStruct((N, D), jnp.float32)] * 5
        + [jax.ShapeDtypeStruct((1, D), jnp.float32)] * 2,
    )(agg_r, agg_d, agg_rc, agg_dc, t1, t2, t3, t4, dvr, dvd,
      b_real.reshape(1, D), a_real.reshape(1, 1),
      b_diff.reshape(1, D), a_diff.reshape(1, 1))

    r_sum, v1, v2 = pl.pallas_call(
        _post_b_body,
        out_shape=[
            jax.ShapeDtypeStruct((1, D), jnp.float32),
            jax.ShapeDtypeStruct((D, 1), jnp.float32),
            jax.ShapeDtypeStruct((D, 1), jnp.float32),
        ],
    )(sr, sd, Wb)

    dcol_spec = pl.BlockSpec((_PB, 1), lambda i: (i, 0))
    vfull_spec = pl.BlockSpec((D, 1), lambda i: (0, 0))
    d1, d2, d3, d4 = pl.pallas_call(
        _post_c_body,
        grid=(nblk,),
        in_specs=[row_spec] * 4 + [vfull_spec, vfull_spec, one_spec],
        out_specs=[dcol_spec] * 4,
        out_shape=[jax.ShapeDtypeStruct((N, 1), jnp.float32)] * 4,
    )(hr, hd, hrc, hdc, v1, v2, bb.reshape(1, 1))
    return d1, d2, d3, d4, r_sum, h_sum


def kernel(x, edge_index, diff_edge_index, diff_edge_weight, corrupted_idx,
           W_real, b_real, a_real, W_diff, b_diff, a_diff, Wb, bb):
    row_r, col_r = edge_index[0], edge_index[1]
    row_d, col_d = diff_edge_index[0], diff_edge_index[1]

    def pad_idx(a):
        return jnp.pad(a.reshape(16, EPT), ((0, 0), (0, PAD_E)),
                       constant_values=N).reshape(16, NCH, 128)

    rowr_p = pad_idx(row_r)
    colr_p = pad_idx(col_r)
    rowd_p = pad_idx(row_d)
    cold_p = pad_idx(col_d)
    w_p = jnp.pad(diff_edge_weight.reshape(16, EPT),
                  ((0, 0), (0, PAD_E))).reshape(16, NCH, 128)

    p_pad = jnp.pad(corrupted_idx, (0, NP - N))
    x_pad = jnp.pad(x, ((0, NP - N), (0, 0)))

    deg, xc = _sc_deg_xc(col_r, col_d, diff_edge_weight, p_pad, x)
    t1, t2, t3, t4, dvr, dvd = _tc_tables(
        x_pad, xc, W_real, W_diff,
        deg[0].reshape(NP, 1), deg[1].reshape(NP, 1))
    agg_r, agg_rc, agg_d, agg_dc = _sc_agg(
        t1, t2, t3, t4, rowr_p, colr_p, rowd_p, cold_p, w_p)

    d1, d2, d3, d4, r_sum, h_sum = _tc_post(
        agg_r[:N], agg_d[:N], agg_rc[:N], agg_dc[:N],
        t1[:N], t2[:N], t3[:N], t4[:N], dvr[:N], dvd[:N],
        b_real, a_real, b_diff, a_diff, Wb, bb)

    disc = jnp.concatenate([d1[:, 0], d2[:, 0], d3[:, 0], d4[:, 0]])
    return (disc, r_sum[0], h_sum)
